# baseline (speedup 1.0000x reference)
import functools

import jax
import jax.numpy as jnp
from jax import lax
from jax.experimental import pallas as pl
from jax.experimental.pallas import tpu as pltpu
from jax.experimental.pallas import tpu_sc as plsc

_H = 16
_S = 2048
_T = 511
_TP = 512
_E = 4096
_PAD0 = (_S - 1) - (_T - 1) // 2
_NSLAB = 6
_NTILE = 16 * _NSLAB + 2


def _body(table_hbm, out_hbm, tcol_v, eext0, bandt, c0s, c510s, sem):
    cid = lax.axis_index("c")
    sid = lax.axis_index("s")
    wid = cid * 16 + sid
    h = wid // 2
    g0 = (wid % 2) * 128

    pltpu.sync_copy(table_hbm.at[pl.ds(pl.multiple_of(h * _TP, 8), _TP)], tcol_v)

    zeros16 = jnp.zeros((16,), jnp.int32)
    c0 = jnp.take(tcol_v[pl.ds(0, 16)], zeros16, axis=0)
    c510 = jnp.take(tcol_v[pl.ds(_TP - 16, 16)], zeros16 + 14, axis=0)

    def fill_head(k, carry):
        eext0[pl.ds(pl.multiple_of(16 * k, 16), 16)] = c0
        return carry

    lax.fori_loop(0, _PAD0 // 16, fill_head, 0)

    def copy_mid(k, carry):
        eext0[pl.ds(pl.multiple_of(_PAD0 + 16 * k, 16), 16)] = tcol_v[
            pl.ds(pl.multiple_of(16 * k, 16), 16)
        ]
        return carry

    lax.fori_loop(0, _TP // 16, copy_mid, 0)

    def fill_tail(k, carry):
        eext0[pl.ds(pl.multiple_of(_PAD0 + _TP + 16 * k, 16), 16)] = c510
        return carry

    lax.fori_loop(0, (_E - _PAD0 - _TP) // 16, fill_tail, 0)

    def fill_slab(t, carry):
        u = t // (_NSLAB * 64)
        rem = t % (_NSLAB * 64)
        j = rem // 64
        r = (rem % 64) // 8
        cc = rem % 8
        q = 8 * u
        klo = (1658 - q) // 128 + 1
        src = 128 * (klo + j) + q + 7 - r + 16 * cc
        bandt[8 * u + r, pl.ds(pl.multiple_of(128 * j + 16 * cc, 16), 16)] = (
            eext0[pl.ds(src, 16)]
        )
        return carry

    lax.fori_loop(0, 16 * _NSLAB * 64, fill_slab, 0)

    def fill_c0(t, carry):
        c0s[t // 32, pl.ds(pl.multiple_of(16 * (t % 32), 16), 16)] = (
            eext0[pl.ds(0, 16)]
        )
        return carry

    lax.fori_loop(0, 8 * 32, fill_c0, 0)

    def fill_c510(t, carry):
        c510s[t // 32, pl.ds(pl.multiple_of(16 * (t % 32), 16), 16)] = (
            eext0[pl.ds(_E - 16, 16)]
        )
        return carry

    lax.fori_loop(0, 8 * 32, fill_c510, 0)

    def row_group(gl, carry):
        g = g0 + gl
        u = (255 - g) % 16
        q = 8 * u
        klo = (1658 - q) // 128 + 1
        kbase = (2040 - 8 * g - q) // 128
        cL = klo - kbase
        cR = cL + _NSLAB - 1
        cLc = jnp.maximum(cL, 0)
        cRc = jnp.minimum(cR, 15)
        interior = jnp.logical_and(cL >= 0, cR <= 15)
        row_u = pl.multiple_of(8 * u, 8)
        dst_rows = pl.ds(pl.multiple_of(8 * g, 8), 8)

        def dst(col_tiles, ntiles):
            return out_hbm.at[
                h, dst_rows, pl.ds(pl.multiple_of(128 * col_tiles, 128), 128 * ntiles)
            ]

        specs = []
        specs.append(
            (interior, bandt.at[pl.ds(row_u, 8), :], dst(cLc, _NSLAB))
        )
        not_int = jnp.logical_not(interior)
        for j in range(_NSLAB):
            cj = cL + j
            cond = jnp.logical_and(not_int, jnp.logical_and(cj >= 0, cj <= 15))
            cjc = jnp.clip(cj, 0, 15)
            specs.append(
                (
                    cond,
                    bandt.at[pl.ds(row_u, 8), pl.ds(128 * j, 128)],
                    dst(cjc, 1),
                )
            )
        for b in range(3):
            specs.append((4 * (b + 1) <= cLc, c0s.at[:, :], dst(4 * b, 4)))
        l4 = 4 * (cLc // 4)
        for t in range(3):
            ct = l4 + t
            specs.append(
                (ct < cLc, c0s.at[:, pl.ds(0, 128)], dst(jnp.clip(ct, 0, 15), 1))
            )
        for b in range(1, 4):
            specs.append((4 * b >= cRc + 1, c510s.at[:, :], dst(4 * b, 4)))
        end4 = 4 * ((cRc + 4) // 4)
        for t in range(3):
            ct = cRc + 1 + t
            cond = jnp.logical_and(ct < end4, ct <= 15)
            specs.append(
                (cond, c510s.at[:, pl.ds(0, 128)], dst(jnp.clip(ct, 0, 15), 1))
            )

        for cond, src, dsl in specs:
            @pl.when(cond)
            def _issue(src=src, dsl=dsl):
                pltpu.async_copy(src, dsl, sem)

        for cond, src, dsl in specs:
            @pl.when(cond)
            def _drain(src=src, dsl=dsl):
                pltpu.make_async_copy(src, dsl, sem).wait()

        return carry

    lax.fori_loop(0, 128, row_group, 0)


def kernel(seq_len, bias_table):
    del seq_len
    tcols = jnp.pad(
        bias_table.astype(jnp.float32).T, ((0, 0), (0, _TP - _T)), mode="edge"
    ).reshape(-1)
    mesh = plsc.VectorSubcoreMesh(core_axis_name="c", subcore_axis_name="s")
    run = functools.partial(
        pl.kernel,
        out_type=jax.ShapeDtypeStruct((_H, _S, _S), jnp.float32),
        mesh=mesh,
        scratch_types=[
            pltpu.VMEM((_TP,), jnp.float32),
            pltpu.VMEM((_E,), jnp.float32),
            pltpu.VMEM((16 * 8, _NSLAB * 128), jnp.float32),
            pltpu.VMEM((8, 512), jnp.float32),
            pltpu.VMEM((8, 512), jnp.float32),
            pltpu.SemaphoreType.DMA,
        ],
    )(_body)
    return run(tcols)


# speedup vs baseline: 1.0181x; 1.0181x over previous
import functools

import jax
import jax.numpy as jnp
from jax import lax
from jax.experimental import pallas as pl
from jax.experimental.pallas import tpu as pltpu
from jax.experimental.pallas import tpu_sc as plsc

_H = 16
_S = 2048
_T = 511
_TP = 512
_E = 4096
_PAD0 = (_S - 1) - (_T - 1) // 2
_NSLAB = 6
_NTILE = 16 * _NSLAB + 2


def _body(table_hbm, out_hbm, tcol_v, eext0, tiles, sem):
    cid = lax.axis_index("c")
    sid = lax.axis_index("s")
    wid = cid * 16 + sid
    h = wid // 2
    g0 = (wid % 2) * 128

    pltpu.sync_copy(table_hbm.at[pl.ds(pl.multiple_of(h * _TP, 8), _TP)], tcol_v)

    zeros16 = jnp.zeros((16,), jnp.int32)
    c0 = jnp.take(tcol_v[pl.ds(0, 16)], zeros16, axis=0)
    c510 = jnp.take(tcol_v[pl.ds(_TP - 16, 16)], zeros16 + 14, axis=0)

    def fill_head(k, carry):
        eext0[pl.ds(pl.multiple_of(16 * k, 16), 16)] = c0
        return carry

    lax.fori_loop(0, _PAD0 // 16, fill_head, 0)

    def copy_mid(k, carry):
        eext0[pl.ds(pl.multiple_of(_PAD0 + 16 * k, 16), 16)] = tcol_v[
            pl.ds(pl.multiple_of(16 * k, 16), 16)
        ]
        return carry

    lax.fori_loop(0, _TP // 16, copy_mid, 0)

    def fill_tail(k, carry):
        eext0[pl.ds(pl.multiple_of(_PAD0 + _TP + 16 * k, 16), 16)] = c510
        return carry

    lax.fori_loop(0, (_E - _PAD0 - _TP) // 16, fill_tail, 0)

    def fill_slab(t, carry):
        u = t // (_NSLAB * 64)
        rem = t % (_NSLAB * 64)
        j = rem // 64
        r = (rem % 64) // 8
        cc = rem % 8
        q = 8 * u
        klo = (1658 - q) // 128 + 1
        src = 128 * (klo + j) + q + 7 - r + 16 * cc
        tiles[(u * _NSLAB + j) * 8 + r, pl.ds(pl.multiple_of(16 * cc, 16), 16)] = (
            eext0[pl.ds(src, 16)]
        )
        return carry

    lax.fori_loop(0, 16 * _NSLAB * 64, fill_slab, 0)

    def fill_c0(t, carry):
        tiles[96 * 8 + t // 8, pl.ds(pl.multiple_of(16 * (t % 8), 16), 16)] = (
            eext0[pl.ds(0, 16)]
        )
        return carry

    lax.fori_loop(0, 64, fill_c0, 0)

    def fill_c510(t, carry):
        tiles[97 * 8 + t // 8, pl.ds(pl.multiple_of(16 * (t % 8), 16), 16)] = (
            eext0[pl.ds(_E - 16, 16)]
        )
        return carry

    lax.fori_loop(0, 64, fill_c510, 0)

    def row_group(gl, carry):
        g = g0 + gl
        u = (255 - g) % 16
        q = 8 * u
        klo = (1658 - q) // 128 + 1
        kbase = (2040 - 8 * g - q) // 128
        copies = []
        for c in range(16):
            j = kbase + c - klo
            tile = jnp.where(j < 0, 96, jnp.where(j > _NSLAB - 1, 97, u * _NSLAB + j))
            copies.append(
                pltpu.async_copy(
                    tiles.at[pl.ds(pl.multiple_of(tile * 8, 8), 8), :],
                    out_hbm.at[
                        h,
                        pl.ds(pl.multiple_of(8 * g, 8), 8),
                        pl.ds(128 * c, 128),
                    ],
                    sem,
                )
            )
        for cp in copies:
            cp.wait()
        return carry

    lax.fori_loop(0, 128, row_group, 0)


def kernel(seq_len, bias_table):
    del seq_len
    tcols = jnp.pad(
        bias_table.astype(jnp.float32).T, ((0, 0), (0, _TP - _T)), mode="edge"
    ).reshape(-1)
    mesh = plsc.VectorSubcoreMesh(core_axis_name="c", subcore_axis_name="s")
    run = functools.partial(
        pl.kernel,
        out_type=jax.ShapeDtypeStruct((_H, _S, _S), jnp.float32),
        mesh=mesh,
        scratch_types=[
            pltpu.VMEM((_TP,), jnp.float32),
            pltpu.VMEM((_E,), jnp.float32),
            pltpu.VMEM((_NTILE * 8, 128), jnp.float32),
            pltpu.SemaphoreType.DMA,
        ],
    )(_body)
    return run(tcols)


# speedup vs baseline: 1.0504x; 1.0318x over previous
import functools

import jax
import jax.numpy as jnp
from jax import lax
from jax.experimental import pallas as pl
from jax.experimental.pallas import tpu as pltpu
from jax.experimental.pallas import tpu_sc as plsc

_H = 16
_S = 2048
_T = 511
_TP = 512
_E = 4096
_PAD0 = (_S - 1) - (_T - 1) // 2
_NSLAB = 6
_NTILE = 16 * _NSLAB + 2


def _body(table_hbm, out_hbm, tcol_v, eext0, tiles, sem):
    cid = lax.axis_index("c")
    sid = lax.axis_index("s")
    wid = cid * 16 + sid
    h = wid // 2
    g0 = (wid % 2) * 128

    pltpu.sync_copy(table_hbm.at[pl.ds(pl.multiple_of(h * _TP, 8), _TP)], tcol_v)

    zeros16 = jnp.zeros((16,), jnp.int32)
    c0 = jnp.take(tcol_v[pl.ds(0, 16)], zeros16, axis=0)
    c510 = jnp.take(tcol_v[pl.ds(_TP - 16, 16)], zeros16 + 14, axis=0)

    def fill_head(k, carry):
        eext0[pl.ds(pl.multiple_of(16 * k, 16), 16)] = c0
        return carry

    lax.fori_loop(0, _PAD0 // 16, fill_head, 0)

    def copy_mid(k, carry):
        eext0[pl.ds(pl.multiple_of(_PAD0 + 16 * k, 16), 16)] = tcol_v[
            pl.ds(pl.multiple_of(16 * k, 16), 16)
        ]
        return carry

    lax.fori_loop(0, _TP // 16, copy_mid, 0)

    def fill_tail(k, carry):
        eext0[pl.ds(pl.multiple_of(_PAD0 + _TP + 16 * k, 16), 16)] = c510
        return carry

    lax.fori_loop(0, (_E - _PAD0 - _TP) // 16, fill_tail, 0)

    def fill_slab(t, carry):
        u = t // (_NSLAB * 64)
        rem = t % (_NSLAB * 64)
        j = rem // 64
        r = (rem % 64) // 8
        cc = rem % 8
        q = 8 * u
        klo = (1658 - q) // 128 + 1
        src = 128 * (klo + j) + q + 7 - r + 16 * cc
        tiles[(u * _NSLAB + j) * 8 + r, pl.ds(pl.multiple_of(16 * cc, 16), 16)] = (
            eext0[pl.ds(src, 16)]
        )
        return carry

    lax.fori_loop(0, 16 * _NSLAB * 64, fill_slab, 0)

    def fill_c0(t, carry):
        tiles[96 * 8 + t // 8, pl.ds(pl.multiple_of(16 * (t % 8), 16), 16)] = (
            eext0[pl.ds(0, 16)]
        )
        return carry

    lax.fori_loop(0, 64, fill_c0, 0)

    def fill_c510(t, carry):
        tiles[97 * 8 + t // 8, pl.ds(pl.multiple_of(16 * (t % 8), 16), 16)] = (
            eext0[pl.ds(_E - 16, 16)]
        )
        return carry

    lax.fori_loop(0, 64, fill_c510, 0)

    def row_group(gl, carry):
        copies = []
        for gg in range(2):
            g = g0 + 2 * gl + gg
            u = (255 - g) % 16
            q = 8 * u
            klo = (1658 - q) // 128 + 1
            kbase = (2040 - 8 * g - q) // 128
            for c in range(16):
                j = kbase + c - klo
                tile = jnp.where(
                    j < 0, 96, jnp.where(j > _NSLAB - 1, 97, u * _NSLAB + j)
                )
                copies.append(
                    pltpu.async_copy(
                        tiles.at[pl.ds(pl.multiple_of(tile * 8, 8), 8), :],
                        out_hbm.at[
                            h,
                            pl.ds(pl.multiple_of(8 * g, 8), 8),
                            pl.ds(128 * c, 128),
                        ],
                        sem,
                    )
                )
        for cp in copies:
            cp.wait()
        return carry

    lax.fori_loop(0, 64, row_group, 0)


def kernel(seq_len, bias_table):
    del seq_len
    tcols = jnp.pad(
        bias_table.astype(jnp.float32).T, ((0, 0), (0, _TP - _T)), mode="edge"
    ).reshape(-1)
    mesh = plsc.VectorSubcoreMesh(core_axis_name="c", subcore_axis_name="s")
    run = functools.partial(
        pl.kernel,
        out_type=jax.ShapeDtypeStruct((_H, _S, _S), jnp.float32),
        mesh=mesh,
        scratch_types=[
            pltpu.VMEM((_TP,), jnp.float32),
            pltpu.VMEM((_E,), jnp.float32),
            pltpu.VMEM((_NTILE * 8, 128), jnp.float32),
            pltpu.SemaphoreType.DMA,
        ],
    )(_body)
    return run(tcols)
